# baseline (speedup 1.0000x reference)
import jax
import jax.numpy as jnp
from jax.experimental import pallas as pl
from jax.experimental.pallas import tpu as pltpu

B, N, D = 8, 2048, 256
CHUNK = 512
NCHUNK = N // CHUNK
BIG = 1 << 30
THRESH2 = 1e-4


def _body(fAB_ref, fBt_ref, fA_ref, fBAt_ref, dA_ref, dB_ref, out_ref):
    a = dA_ref[0]
    b = dB_ref[0]
    na = a * (20.0 / jnp.sqrt(jnp.sum(a * a, axis=-1, keepdims=True)))
    nb = b / jnp.sqrt(jnp.sum(b * b, axis=-1, keepdims=True))

    fBt = fBt_ref[0]
    fBAt = fBAt_ref[0]

    se_c = jnp.zeros((1, N), jnp.float32)
    min_A = jnp.full((1, N), jnp.inf, jnp.float32)
    arg_A = jnp.full((1, N), BIG, jnp.int32)
    se_r_chunks = []
    min_B_chunks = []
    jstar_chunks = []
    corrsel_chunks = []
    for ci in range(NCHUNK):
        r0 = ci * CHUNK
        corr = jax.lax.dot_general(
            na[r0:r0 + CHUNK], nb, (((1,), (1,)), ((), ())),
            preferred_element_type=jnp.float32)
        e = jnp.exp(corr)
        se_r_chunks.append(jnp.sum(e, axis=1, keepdims=True))
        se_c = se_c + jnp.sum(e, axis=0, keepdims=True)

        iota_m = jax.lax.broadcasted_iota(jnp.int32, (CHUNK, N), 1)
        iota_n = jax.lax.broadcasted_iota(jnp.int32, (CHUNK, N), 0) + r0

        dx = fAB_ref[0, r0:r0 + CHUNK, 0:1] - fBt[0:1, :]
        dy = fAB_ref[0, r0:r0 + CHUNK, 1:2] - fBt[1:2, :]
        d2B = dx * dx + dy * dy
        mB = jnp.min(d2B, axis=1, keepdims=True)
        min_B_chunks.append(mB)
        selB = d2B == mB
        js = jnp.min(jnp.where(selB, iota_m, BIG), axis=1,
                     keepdims=True)
        jstar_chunks.append(js)
        corrsel_chunks.append(jnp.max(
            jnp.where(selB, corr, -jnp.inf), axis=1, keepdims=True))

        dx = fA_ref[0, r0:r0 + CHUNK, 0:1] - fBAt[0:1, :]
        dy = fA_ref[0, r0:r0 + CHUNK, 1:2] - fBAt[1:2, :]
        d2A = dx * dx + dy * dy
        cmin = jnp.min(d2A, axis=0, keepdims=True)
        carg = jnp.min(jnp.where(d2A == cmin, iota_n, BIG), axis=0,
                       keepdims=True)
        upd = cmin < min_A
        arg_A = jnp.where(upd, carg, arg_A)
        min_A = jnp.minimum(min_A, cmin)

    lse_c = jnp.log(se_c)

    num = jnp.float32(0.0)
    cnt = jnp.float32(0.0)
    for ci in range(NCHUNK):
        r0 = ci * CHUNK
        iota_m = jax.lax.broadcasted_iota(jnp.int32, (CHUNK, N), 1)
        iota_n = jax.lax.broadcasted_iota(jnp.int32, (CHUNK, N), 0) + r0
        sel = (iota_m == jstar_chunks[ci]) & (arg_A == iota_n)
        ok = (sel & (min_B_chunks[ci] < THRESH2) & (min_A < THRESH2))
        lse_r = jnp.log(se_r_chunks[ci])
        val = (2.0 * corrsel_chunks[ci] - lse_r) - lse_c
        num = num + jnp.sum(jnp.where(ok, val, 0.0))
        cnt = cnt + jnp.sum(ok.astype(jnp.float32))

    lane = jax.lax.broadcasted_iota(jnp.int32, (1, 1, 128), 2)
    out_ref[...] = jnp.where(lane == 0, num, cnt)


@jax.jit
def kernel(kpts_A, kpts_B, kpts_A_to_B, kpts_B_to_A,
           descriptions_A, descriptions_B):
    fAB = kpts_A_to_B
    fBt = jnp.swapaxes(kpts_B, 1, 2)
    fA = kpts_A
    fBAt = jnp.swapaxes(kpts_B_to_A, 1, 2)

    batch_spec = lambda shp: pl.BlockSpec((1,) + shp, lambda i: (i, 0, 0))
    out = pl.pallas_call(
        _body,
        grid=(B,),
        in_specs=[
            batch_spec((N, 2)),
            batch_spec((2, N)),
            batch_spec((N, 2)),
            batch_spec((2, N)),
            batch_spec((N, D)),
            batch_spec((N, D)),
        ],
        out_specs=pl.BlockSpec((1, 1, 128), lambda i: (i, 0, 0)),
        out_shape=jax.ShapeDtypeStruct((B, 1, 128), jnp.float32),
    )(fAB, fBt, fA, fBAt, descriptions_A, descriptions_B)

    total_num = jnp.sum(out[:, 0, 0])
    total_cnt = jnp.sum(out[:, 0, 1])
    return -total_num / jnp.maximum(total_cnt, 1.0)


# speedup vs baseline: 1.2271x; 1.2271x over previous
import jax
import jax.numpy as jnp
from jax.experimental import pallas as pl
from jax.experimental.pallas import tpu as pltpu

B, N, D = 8, 2048, 256
CHUNK = 512
NCHUNK = N // CHUNK
BIG = 1 << 30
THRESH2 = 1e-4


def _body(kAB_ref, kBt_ref, kBA_ref, kAt_ref, dA_ref, dB_ref, out_ref):
    a = dA_ref[0]
    b = dB_ref[0]
    na = a * (20.0 / jnp.sqrt(jnp.sum(a * a, axis=-1, keepdims=True)))
    nb = b / jnp.sqrt(jnp.sum(b * b, axis=-1, keepdims=True))

    kBt = kBt_ref[0]
    kAt = kAt_ref[0]

    se_c = jnp.zeros((1, N), jnp.float32)
    se_r_chunks = []
    min_B_chunks = []
    jstar_chunks = []
    corrsel_chunks = []
    min_A_chunks = []
    istar_chunks = []
    for ci in range(NCHUNK):
        r0 = ci * CHUNK
        corr = jax.lax.dot_general(
            na[r0:r0 + CHUNK], nb, (((1,), (1,)), ((), ())),
            preferred_element_type=jnp.float32)
        e = jnp.exp(corr)
        se_r_chunks.append(jnp.sum(e, axis=1, keepdims=True))
        se_c = se_c + jnp.sum(e, axis=0, keepdims=True)

        iota_m = jax.lax.broadcasted_iota(jnp.int32, (CHUNK, N), 1)

        dx = kAB_ref[0, r0:r0 + CHUNK, 0:1] - kBt[0:1, :]
        dy = kAB_ref[0, r0:r0 + CHUNK, 1:2] - kBt[1:2, :]
        d2B = dx * dx + dy * dy
        mB = jnp.min(d2B, axis=1, keepdims=True)
        min_B_chunks.append(mB)
        selB = d2B == mB
        jstar_chunks.append(jnp.min(jnp.where(selB, iota_m, BIG), axis=1,
                                    keepdims=True))
        corrsel_chunks.append(jnp.max(
            jnp.where(selB, corr, -jnp.inf), axis=1, keepdims=True))

        dx = kBA_ref[0, r0:r0 + CHUNK, 0:1] - kAt[0:1, :]
        dy = kBA_ref[0, r0:r0 + CHUNK, 1:2] - kAt[1:2, :]
        d2A = dx * dx + dy * dy
        mA = jnp.min(d2A, axis=1, keepdims=True)
        min_A_chunks.append(mA)
        istar_chunks.append(jnp.min(jnp.where(d2A == mA, iota_m, BIG),
                                    axis=1, keepdims=True))

    lse_c = jnp.log(se_c)
    istar_f = jnp.concatenate(istar_chunks, axis=0).astype(jnp.float32)
    min_A = jnp.concatenate(min_A_chunks, axis=0)
    stacked = jnp.concatenate(
        [istar_f, min_A, jnp.zeros((N, 6), jnp.float32)], axis=1)

    num = jnp.float32(0.0)
    cnt = jnp.float32(0.0)
    matched = jnp.zeros((1, N), jnp.float32)
    for ci in range(NCHUNK):
        r0 = ci * CHUNK
        iota_m = jax.lax.broadcasted_iota(jnp.int32, (CHUNK, N), 1)
        onehot = (iota_m == jstar_chunks[ci]).astype(jnp.float32)
        g = jnp.dot(onehot, stacked, preferred_element_type=jnp.float32)
        rowf = (jax.lax.broadcasted_iota(jnp.int32, (CHUNK, 1), 0)
                + r0).astype(jnp.float32)
        mutual = ((g[:, 0:1] == rowf) & (min_B_chunks[ci] < THRESH2)
                  & (g[:, 1:2] < THRESH2))
        mutf = mutual.astype(jnp.float32)
        lse_r = jnp.log(se_r_chunks[ci])
        num = num + jnp.sum(mutf * (2.0 * corrsel_chunks[ci] - lse_r))
        cnt = cnt + jnp.sum(mutf)
        matched = matched + jax.lax.dot_general(
            mutf, onehot, (((0,), (0,)), ((), ())),
            preferred_element_type=jnp.float32)
    num = num - jnp.sum(matched * lse_c)

    lane = jax.lax.broadcasted_iota(jnp.int32, (1, 1, 128), 2)
    out_ref[...] = jnp.where(lane == 0, num, cnt)


@jax.jit
def kernel(kpts_A, kpts_B, kpts_A_to_B, kpts_B_to_A,
           descriptions_A, descriptions_B):
    kBt = jnp.swapaxes(kpts_B, 1, 2)
    kAt = jnp.swapaxes(kpts_A, 1, 2)

    batch_spec = lambda shp: pl.BlockSpec((1,) + shp, lambda i: (i, 0, 0))
    out = pl.pallas_call(
        _body,
        grid=(B,),
        in_specs=[
            batch_spec((N, 2)),
            batch_spec((2, N)),
            batch_spec((N, 2)),
            batch_spec((2, N)),
            batch_spec((N, D)),
            batch_spec((N, D)),
        ],
        out_specs=pl.BlockSpec((1, 1, 128), lambda i: (i, 0, 0)),
        out_shape=jax.ShapeDtypeStruct((B, 1, 128), jnp.float32),
    )(kpts_A_to_B, kBt, kpts_B_to_A, kAt, descriptions_A, descriptions_B)

    total_num = jnp.sum(out[:, 0, 0])
    total_cnt = jnp.sum(out[:, 0, 1])
    return -total_num / jnp.maximum(total_cnt, 1.0)
